# baseline (speedup 1.0000x reference)
import functools

import jax
import jax.numpy as jnp
from jax import lax
from jax.experimental import pallas as pl
from jax.experimental.pallas import tpu as pltpu

K = 16
N = 2048
P = 64
BS = 8
RB = 1024
NB = N // RB


def _topk_head(a_ref, r_blk, w2_ref, b2_ref, tgt_row, out_ref):
    neg = jnp.float32(-jnp.inf)
    a_blk = a_ref[...]
    t = jnp.max(a_blk, axis=1, keepdims=True)
    for _ in range(K - 1):
        t = jnp.max(jnp.where(a_blk < t, a_blk, neg), axis=1, keepdims=True)
    sel = jnp.where(a_blk >= t, 1.0, 0.0)

    pooled = jnp.dot(sel, r_blk,
                     preferred_element_type=jnp.float32) * (1.0 / K)
    out = jnp.dot(pooled, w2_ref[...],
                  preferred_element_type=jnp.float32) + b2_ref[...]
    out_ref[0] = out + tgt_row


def _corr_mm(xn_s, row_start, batch_start, prec):
    rows = xn_s[pl.ds(row_start, RB), :]
    cols = xn_s[pl.ds(batch_start, N), :]
    return lax.dot_general(rows, cols, (((1,), (1,)), ((), ())),
                           precision=prec,
                           preferred_element_type=jnp.float32)


def _fused_body(xs_ref, tgtv_ref, w1_ref, b1_ref, w2_ref, b2_ref,
                out_ref, tgt_ref, xn_s, r_s, a0_s, a1_s, prec):
    b = pl.program_id(0)
    i = pl.program_id(1)

    @pl.when((b == 0) & (i == 0))
    def _prologue():
        xs = xs_ref[...].reshape(BS * N, P)
        xm = xs - jnp.mean(xs, axis=-1, keepdims=True)
        denom = jnp.sqrt(jnp.sum(xm * xm, axis=-1, keepdims=True)) + 1e-8
        xn_s[...] = xm / denom
        h = jnp.dot(xs, w1_ref[...], preferred_element_type=jnp.float32)
        r_s[...] = jax.nn.relu(h + b1_ref[...])
        a0_s[...] = _corr_mm(xn_s, 0, 0, prec)

    @pl.when(i == 0)
    def _prefetch_second_half():
        a1_s[...] = _corr_mm(xn_s, b * N + RB, b * N, prec)

    @pl.when((i == 1) & (b < BS - 1))
    def _prefetch_next_batch():
        a0_s[...] = _corr_mm(xn_s, (b + 1) * N, (b + 1) * N, prec)

    tgt_row = jnp.mean(tgtv_ref[0], axis=0, keepdims=True)
    tgt_ref[0] = tgt_row

    r_blk = r_s[pl.ds(b * N, N), :]

    @pl.when(i == 0)
    def _first():
        _topk_head(a0_s, r_blk, w2_ref, b2_ref, tgt_row, out_ref)

    @pl.when(i == 1)
    def _second():
        _topk_head(a1_s, r_blk, w2_ref, b2_ref, tgt_row, out_ref)


@functools.partial(jax.jit, static_argnames=())
def _run(xs, tgtv, W1, b1, W2, b2):
    body = functools.partial(_fused_body, prec=lax.Precision.DEFAULT)
    out, tgt2d = pl.pallas_call(
        body,
        grid=(BS, NB),
        in_specs=[
            pl.BlockSpec((BS, N, P), lambda b, i: (0, 0, 0)),
            pl.BlockSpec((1, 12, P), lambda b, i: (b, 0, 0)),
            pl.BlockSpec((P, P), lambda b, i: (0, 0)),
            pl.BlockSpec((1, P), lambda b, i: (0, 0)),
            pl.BlockSpec((P, P), lambda b, i: (0, 0)),
            pl.BlockSpec((1, P), lambda b, i: (0, 0)),
        ],
        out_specs=[
            pl.BlockSpec((1, RB, P), lambda b, i: (b, i, 0)),
            pl.BlockSpec((1, 1, P), lambda b, i: (b, 0, 0)),
        ],
        out_shape=[
            jax.ShapeDtypeStruct((BS, N, P), jnp.float32),
            jax.ShapeDtypeStruct((BS, 1, P), jnp.float32),
        ],
        scratch_shapes=[
            pltpu.VMEM((BS * N, P), jnp.float32),
            pltpu.VMEM((BS * N, P), jnp.float32),
            pltpu.VMEM((RB, N), jnp.float32),
            pltpu.VMEM((RB, N), jnp.float32),
        ],
    )(xs, tgtv, W1, b1, W2, b2)
    return out, tgt2d


def kernel(x_pr, x_p, tgt_mode, mode, number, W1, b1, W2, b2):
    xs = jnp.transpose(jnp.take(x_pr, number, axis=2), (0, 2, 1))
    xs = xs.astype(jnp.float32)
    tgtv = jnp.take(x_p, number, axis=3).astype(jnp.float32)
    out, tgt2d = _run(xs, tgtv, W1, b1.reshape(1, P), W2, b2.reshape(1, P))
    sq_pr = out
    tgt_out = tgt2d.reshape(BS, P, 1)
    return (sq_pr, tgt_out)


# speedup vs baseline: 2.0578x; 2.0578x over previous
import functools

import jax
import jax.numpy as jnp
from jax import lax
from jax.experimental import pallas as pl
from jax.experimental.pallas import tpu as pltpu

K = 16
N = 2048
P = 64
BS = 8
RB = 1024
NB = N // RB


def _normalize(xs):
    xm = xs - jnp.mean(xs, axis=-1, keepdims=True)
    denom = jnp.sqrt(jnp.sum(xm * xm, axis=-1, keepdims=True)) + 1e-8
    return xm / denom


def _corr_mm(rows, cols, prec):
    return lax.dot_general(rows, cols, (((1,), (1,)), ((), ())),
                           precision=prec,
                           preferred_element_type=jnp.float32)


def _topk_head(a_ref, r_ref, w2_ref, b2_ref, tgt_row, out_ref):
    neg = jnp.float32(-jnp.inf)
    a_blk = a_ref[...]
    t = jnp.max(a_blk, axis=1, keepdims=True)
    for _ in range(K - 1):
        t = jnp.max(jnp.where(a_blk < t, a_blk, neg), axis=1, keepdims=True)
    sel = jnp.where(a_blk >= t, 1.0, 0.0)

    pooled = jnp.dot(sel, r_ref[...],
                     preferred_element_type=jnp.float32) * (1.0 / K)
    out = jnp.dot(pooled, w2_ref[...],
                  preferred_element_type=jnp.float32) + b2_ref[...]
    out_ref[0] = out + tgt_row


def _fused_body(xs_ref, tgtv_ref, w1_ref, b1_ref, w2_ref, b2_ref,
                out_ref, tgt_ref, xn0_s, xn1_s, rb_s, a0_s, a1_s, prec):
    b = pl.program_id(0)
    i = pl.program_id(1)

    @pl.when((b == 0) & (i == 0))
    def _prologue0():
        xn = _normalize(xs_ref[0])
        xn0_s[...] = xn
        h = jnp.dot(xs_ref[0], w1_ref[...],
                    preferred_element_type=jnp.float32)
        rb_s[...] = jax.nn.relu(h + b1_ref[...])
        a0_s[...] = _corr_mm(xn[:RB], xn, prec)

    @pl.when((i == 0) & (b % 2 == 0))
    def _mm1_even():
        a1_s[...] = _corr_mm(xn0_s[RB:], xn0_s[...], prec)

    @pl.when((i == 0) & (b % 2 == 1))
    def _mm1_odd():
        a1_s[...] = _corr_mm(xn1_s[RB:], xn1_s[...], prec)

    @pl.when((i == 1) & (b < BS - 1) & (b % 2 == 1))
    def _pre_even():
        xn = _normalize(xs_ref[0])
        xn0_s[...] = xn
        a0_s[...] = _corr_mm(xn[:RB], xn, prec)

    @pl.when((i == 1) & (b < BS - 1) & (b % 2 == 0))
    def _pre_odd():
        xn = _normalize(xs_ref[0])
        xn1_s[...] = xn
        a0_s[...] = _corr_mm(xn[:RB], xn, prec)

    tgt_row = jnp.mean(tgtv_ref[0], axis=0, keepdims=True)
    tgt_ref[0] = tgt_row

    @pl.when(i == 0)
    def _first():
        _topk_head(a0_s, rb_s, w2_ref, b2_ref, tgt_row, out_ref)

    @pl.when(i == 1)
    def _second():
        _topk_head(a1_s, rb_s, w2_ref, b2_ref, tgt_row, out_ref)

    @pl.when((i == 1) & (b < BS - 1))
    def _project_next():
        h = jnp.dot(xs_ref[0], w1_ref[...],
                    preferred_element_type=jnp.float32)
        rb_s[...] = jax.nn.relu(h + b1_ref[...])


@functools.partial(jax.jit, static_argnames=())
def _run(xs, tgtv, W1, b1, W2, b2):
    body = functools.partial(_fused_body, prec=lax.Precision.DEFAULT)
    out, tgt2d = pl.pallas_call(
        body,
        grid=(BS, NB),
        in_specs=[
            pl.BlockSpec((1, N, P),
                         lambda b, i: (jnp.minimum(b + i, BS - 1), 0, 0)),
            pl.BlockSpec((1, 12, P), lambda b, i: (b, 0, 0)),
            pl.BlockSpec((P, P), lambda b, i: (0, 0)),
            pl.BlockSpec((1, P), lambda b, i: (0, 0)),
            pl.BlockSpec((P, P), lambda b, i: (0, 0)),
            pl.BlockSpec((1, P), lambda b, i: (0, 0)),
        ],
        out_specs=[
            pl.BlockSpec((1, RB, P), lambda b, i: (b, i, 0)),
            pl.BlockSpec((1, 1, P), lambda b, i: (b, 0, 0)),
        ],
        out_shape=[
            jax.ShapeDtypeStruct((BS, N, P), jnp.float32),
            jax.ShapeDtypeStruct((BS, 1, P), jnp.float32),
        ],
        scratch_shapes=[
            pltpu.VMEM((N, P), jnp.float32),
            pltpu.VMEM((N, P), jnp.float32),
            pltpu.VMEM((N, P), jnp.float32),
            pltpu.VMEM((RB, N), jnp.float32),
            pltpu.VMEM((RB, N), jnp.float32),
        ],
    )(xs, tgtv, W1, b1, W2, b2)
    return out, tgt2d


def kernel(x_pr, x_p, tgt_mode, mode, number, W1, b1, W2, b2):
    xs = jnp.transpose(jnp.take(x_pr, number, axis=2), (0, 2, 1))
    xs = xs.astype(jnp.float32)
    tgtv = jnp.take(x_p, number, axis=3).astype(jnp.float32)
    out, tgt2d = _run(xs, tgtv, W1, b1.reshape(1, P), W2, b2.reshape(1, P))
    sq_pr = out
    tgt_out = tgt2d.reshape(BS, P, 1)
    return (sq_pr, tgt_out)
